# baseline (speedup 1.0000x reference)
import functools
import math

import jax
import jax.numpy as jnp
import numpy as np
from jax.experimental import pallas as pl
from jax.experimental.pallas import tpu as pltpu

B, S, D, H = 2, 512, 768, 12
SAMPLE_NUM, MU_NUM, K_NN = 2, 1, 16
MARGIN, GAMMA = 10.0, 0.5
DH = D // H
N1 = (MU_NUM + SAMPLE_NUM) * S
N2 = 2 * N1
BR = 128
NB = N2 // BR
_ENT_CONST = D / 2.0 * (math.log(2.0 * math.pi) + 1.0)


def _make_noise() -> np.ndarray:
    key = jax.random.key(42)
    out = np.zeros((2, B, SAMPLE_NUM, S, D), np.float32)
    for b in range(B):
        for s in range(SAMPLE_NUM):
            for m in range(2):
                k = jax.random.fold_in(key, b * 100 + s + 50 * m)
                out[m, b, s] = np.asarray(jax.random.normal(k, (S, D), jnp.float32))
    return out


_NOISE = _make_noise()


def _enc_kernel(x_ref, wq_ref, wk_ref, wv_ref, wo_ref, wmu_ref, wls_ref,
                n_ref, nodes_ref, stats_ref):
    x = x_ref[0, 0]
    q = x @ wq_ref[0]
    k = x @ wk_ref[0]
    v = x @ wv_ref[0]
    scale = 1.0 / math.sqrt(DH)
    acc = jnp.zeros((S, D), jnp.float32)
    wo = wo_ref[0]
    for h in range(H):
        qh = q[:, h * DH:(h + 1) * DH]
        kh = k[:, h * DH:(h + 1) * DH]
        vh = v[:, h * DH:(h + 1) * DH]
        logits = jax.lax.dot_general(qh, kh, (((1,), (1,)), ((), ())),
                                     preferred_element_type=jnp.float32) * scale
        m = jnp.max(logits, axis=-1, keepdims=True)
        e = jnp.exp(logits - m)
        att = e / jnp.sum(e, axis=-1, keepdims=True)
        acc += (att @ vh) @ wo[h * DH:(h + 1) * DH, :]
    hid = acc + x
    mu = hid @ wmu_ref[0]
    ls = hid @ wls_ref[0]
    e_ls = jnp.exp(ls)
    nodes_ref[0, 0, 0:S, :] = mu
    nodes_ref[0, 0, S:2 * S, :] = mu + e_ls * n_ref[0, 0, 0]
    nodes_ref[0, 0, 2 * S:3 * S, :] = mu + e_ls * n_ref[0, 0, 1]
    max_mu = jnp.max(mu)
    max_ls = jnp.max(ls)
    entropy = _ENT_CONST + jnp.sum(ls, axis=1) / 2.0
    mel = jnp.mean(jnp.maximum(MARGIN - entropy, 0.0))
    lane = jax.lax.broadcasted_iota(jnp.int32, (1, 1, 1, 128), 3)
    stats_ref[...] = jnp.where(lane == 0, max_mu,
                               jnp.where(lane == 1, max_ls, mel))


def _encode(x_all, weights, noise):
    grid = (2, B)
    wspec = pl.BlockSpec((1, D, D), lambda m, b: (m, 0, 0))
    out = pl.pallas_call(
        _enc_kernel,
        grid=grid,
        in_specs=[
            pl.BlockSpec((1, 1, S, D), lambda m, b: (m, b, 0, 0)),
            wspec, wspec, wspec, wspec, wspec, wspec,
            pl.BlockSpec((1, 1, SAMPLE_NUM, S, D), lambda m, b: (m, b, 0, 0, 0)),
        ],
        out_specs=[
            pl.BlockSpec((1, 1, N1, D), lambda m, b: (m, b, 0, 0)),
            pl.BlockSpec((1, 1, 1, 128), lambda m, b: (m, b, 0, 0)),
        ],
        out_shape=[
            jax.ShapeDtypeStruct((2, B, N1, D), jnp.float32),
            jax.ShapeDtypeStruct((2, B, 1, 128), jnp.float32),
        ],
    )(x_all, *weights, noise)
    return out


def _top16_thr(sim):
    thr = jnp.max(sim, axis=1, keepdims=True)
    for _ in range(K_NN - 1):
        thr = jnp.max(jnp.where(sim < thr, sim, -jnp.inf), axis=1, keepdims=True)
    return thr


def _gcn1_kernel(nodes_ref, w1m_ref, w1te_ref,
                 h1m_ref, maskm_ref, h1te_ref, maskte_ref):
    nb = pl.program_id(1)
    nodes = nodes_ref[0]
    rows = nodes_ref[0, pl.ds(nb * BR, BR), :]
    sim = jax.lax.dot_general(rows, nodes, (((1,), (1,)), ((), ())),
                              preferred_element_type=jnp.float32)
    nodes_bf = nodes.astype(jnp.bfloat16)

    mask_m = (sim >= _top16_thr(sim)).astype(jnp.bfloat16)
    maskm_ref[0] = mask_m
    agg_m = jax.lax.dot_general(mask_m, nodes_bf, (((1,), (0,)), ((), ())),
                                preferred_element_type=jnp.float32) * (1.0 / K_NN)
    h1m_ref[0] = jnp.maximum(agg_m @ w1m_ref[0], 0.0).astype(jnp.bfloat16)

    base = jnp.where(nb < NB // 2, 0, N1)
    sim_sub = jnp.where(nb < NB // 2, sim[:, :N1], sim[:, N1:])
    half_nodes = nodes_ref[0, pl.ds(base, N1), :].astype(jnp.bfloat16)
    mask_te = (sim_sub >= _top16_thr(sim_sub)).astype(jnp.bfloat16)
    maskte_ref[0] = mask_te
    agg_te = jax.lax.dot_general(mask_te, half_nodes, (((1,), (0,)), ((), ())),
                                 preferred_element_type=jnp.float32) * (1.0 / K_NN)
    h1te_ref[0] = jnp.maximum(agg_te @ w1te_ref[0], 0.0).astype(jnp.bfloat16)


def _gcn2_kernel(maskm_ref, h1m_ref, maskte_ref, h1te_ref,
                 w2m_ref, wcm_ref, w2te_ref, wcte_ref,
                 outm_ref, outte_ref, accm_ref, accte_ref):
    nb = pl.program_id(1)

    @pl.when(nb == 0)
    def _():
        accm_ref[...] = jnp.zeros_like(accm_ref)

    @pl.when((nb == 0) | (nb == NB // 2))
    def _():
        accte_ref[...] = jnp.zeros_like(accte_ref)

    agg2_m = jax.lax.dot_general(maskm_ref[0], h1m_ref[0], (((1,), (0,)), ((), ())),
                                 preferred_element_type=jnp.float32) * (1.0 / K_NN)
    h2_m = jnp.maximum(agg2_m @ w2m_ref[0], 0.0)
    accm_ref[...] += jnp.sum(h2_m, axis=0, keepdims=True)

    agg2_te = jax.lax.dot_general(maskte_ref[0], h1te_ref[0], (((1,), (0,)), ((), ())),
                                  preferred_element_type=jnp.float32) * (1.0 / K_NN)
    h2_te = jnp.maximum(agg2_te @ w2te_ref[0], 0.0)
    accte_ref[...] += jnp.sum(h2_te, axis=0, keepdims=True)

    @pl.when(nb == NB // 2 - 1)
    def _():
        outte_ref[0, 0:1, :] = (accte_ref[...] * (1.0 / N1)) @ wcte_ref[0]

    @pl.when(nb == NB - 1)
    def _():
        outte_ref[0, 1:2, :] = (accte_ref[...] * (1.0 / N1)) @ wcte_ref[0]
        outm_ref[0] = (accm_ref[...] * (1.0 / N2)) @ wcm_ref[0]


def _gcn_all(m_nodes, w1m, w2m, wcm, w1te, w2te, wcte):
    half = lambda b, nb: (nb // (NB // 2), 0, 0)
    h1m, maskm, h1te, maskte = pl.pallas_call(
        _gcn1_kernel,
        grid=(B, NB),
        in_specs=[
            pl.BlockSpec((1, N2, D), lambda b, nb: (b, 0, 0)),
            pl.BlockSpec((1, D, 512), lambda b, nb: (0, 0, 0)),
            pl.BlockSpec((1, D, 512), half),
        ],
        out_specs=[
            pl.BlockSpec((1, BR, 512), lambda b, nb: (b, nb, 0)),
            pl.BlockSpec((1, BR, N2), lambda b, nb: (b, nb, 0)),
            pl.BlockSpec((1, BR, 512), lambda b, nb: (b, nb, 0)),
            pl.BlockSpec((1, BR, N1), lambda b, nb: (b, nb, 0)),
        ],
        out_shape=[
            jax.ShapeDtypeStruct((B, N2, 512), jnp.bfloat16),
            jax.ShapeDtypeStruct((B, N2, N2), jnp.bfloat16),
            jax.ShapeDtypeStruct((B, N2, 512), jnp.bfloat16),
            jax.ShapeDtypeStruct((B, N2, N1), jnp.bfloat16),
        ],
    )(m_nodes, w1m[None], jnp.stack([w1te[0], w1te[1]]))
    outm, outte = pl.pallas_call(
        _gcn2_kernel,
        grid=(B, NB),
        in_specs=[
            pl.BlockSpec((1, BR, N2), lambda b, nb: (b, nb, 0)),
            pl.BlockSpec((1, N2, 512), lambda b, nb: (b, 0, 0)),
            pl.BlockSpec((1, BR, N1), lambda b, nb: (b, nb, 0)),
            pl.BlockSpec((1, N1, 512), lambda b, nb: (b, nb // (NB // 2), 0)),
            pl.BlockSpec((1, 512, 256), lambda b, nb: (0, 0, 0)),
            pl.BlockSpec((1, 256, 2), lambda b, nb: (0, 0, 0)),
            pl.BlockSpec((1, 512, 256), half),
            pl.BlockSpec((1, 256, 2), half),
        ],
        out_specs=[
            pl.BlockSpec((1, 1, 2), lambda b, nb: (b, 0, 0)),
            pl.BlockSpec((1, 2, 2), lambda b, nb: (b, 0, 0)),
        ],
        out_shape=[
            jax.ShapeDtypeStruct((B, 1, 2), jnp.float32),
            jax.ShapeDtypeStruct((B, 2, 2), jnp.float32),
        ],
        scratch_shapes=[pltpu.VMEM((1, 256), jnp.float32),
                        pltpu.VMEM((1, 256), jnp.float32)],
    )(maskm, h1m, maskte, h1te, w2m[None], wcm[None],
      jnp.stack([w2te[0], w2te[1]]), jnp.stack([wcte[0], wcte[1]]))
    return outm[:, 0, :], outte


def kernel(img_embeds, text_embeds,
           img_Wq, img_Wk, img_Wv, img_Wo, img_Wmu, img_Wls,
           txt_Wq, txt_Wk, txt_Wv, txt_Wo, txt_Wmu, txt_Wls,
           tcls_W1, tcls_W2, tcls_Wc,
           icls_W1, icls_W2, icls_Wc,
           hcls_W1, hcls_W2, hcls_Wc):
    x_all = jnp.stack([img_embeds, text_embeds])
    weights = [jnp.stack([i, t]) for i, t in [
        (img_Wq, txt_Wq), (img_Wk, txt_Wk), (img_Wv, txt_Wv),
        (img_Wo, txt_Wo), (img_Wmu, txt_Wmu), (img_Wls, txt_Wls)]]
    noise = jnp.asarray(_NOISE)
    nodes, stats = _encode(x_all, weights, noise)
    m_nodes = jnp.concatenate([nodes[1], nodes[0]], axis=1)

    deci_m, deci_te = _gcn_all(
        m_nodes, hcls_W1, hcls_W2, hcls_Wc,
        (tcls_W1, icls_W1), (tcls_W2, icls_W2), (tcls_Wc, icls_Wc))
    deci_t, deci_e = deci_te[:, 0, :], deci_te[:, 1, :]

    max_mu = stats[..., 0, 0]
    max_ls = stats[..., 0, 1]
    mel = stats[..., 0, 2]
    std = jnp.sqrt(jnp.maximum(max_ls, 0.0) + 1e-06)
    score = jax.nn.sigmoid(std / max_mu)
    img_score, txt_score = score[0], score[1]
    m_ab = txt_score * img_score
    margin_loss = (mel[0, B - 1] + mel[1, B - 1]) / 2.0
    preds = jnp.where((m_ab > GAMMA)[:, None], deci_m,
                      jnp.where((txt_score > img_score)[:, None], deci_t, deci_e))
    return preds, margin_loss


# speedup vs baseline: 1.2119x; 1.2119x over previous
import functools
import math

import jax
import jax.numpy as jnp
import numpy as np
from jax.experimental import pallas as pl
from jax.experimental.pallas import tpu as pltpu

B, S, D, H = 2, 512, 768, 12
SAMPLE_NUM, MU_NUM, K_NN = 2, 1, 16
MARGIN, GAMMA = 10.0, 0.5
DH = D // H
N1 = (MU_NUM + SAMPLE_NUM) * S
N2 = 2 * N1
BR = 256
NB = N2 // BR
_ENT_CONST = D / 2.0 * (math.log(2.0 * math.pi) + 1.0)


def _make_noise() -> np.ndarray:
    key = jax.random.key(42)
    out = np.zeros((2, B, SAMPLE_NUM, S, D), np.float32)
    for b in range(B):
        for s in range(SAMPLE_NUM):
            for m in range(2):
                k = jax.random.fold_in(key, b * 100 + s + 50 * m)
                out[m, b, s] = np.asarray(jax.random.normal(k, (S, D), jnp.float32))
    return out


_NOISE = _make_noise()


def _enc_kernel(x_ref, wq_ref, wk_ref, wv_ref, wo_ref, wmu_ref, wls_ref,
                n_ref, nodes_ref, stats_ref):
    x = x_ref[0, 0]
    q = x @ wq_ref[0]
    k = x @ wk_ref[0]
    v = x @ wv_ref[0]
    scale = 1.0 / math.sqrt(DH)
    heads = []
    for h in range(H):
        qh = q[:, h * DH:(h + 1) * DH]
        kh = k[:, h * DH:(h + 1) * DH]
        vh = v[:, h * DH:(h + 1) * DH]
        logits = jax.lax.dot_general(qh, kh, (((1,), (1,)), ((), ())),
                                     preferred_element_type=jnp.float32) * scale
        m = jnp.max(logits, axis=-1, keepdims=True)
        e = jnp.exp(logits - m)
        att = e / jnp.sum(e, axis=-1, keepdims=True)
        heads.append(att @ vh)
    hid = jnp.concatenate(heads, axis=1) @ wo_ref[0] + x
    mu = hid @ wmu_ref[0]
    ls = hid @ wls_ref[0]
    e_ls = jnp.exp(ls)
    nodes_ref[0, 0, 0:S, :] = mu
    nodes_ref[0, 0, S:2 * S, :] = mu + e_ls * n_ref[0, 0, 0]
    nodes_ref[0, 0, 2 * S:3 * S, :] = mu + e_ls * n_ref[0, 0, 1]
    max_mu = jnp.max(mu)
    max_ls = jnp.max(ls)
    entropy = _ENT_CONST + jnp.sum(ls, axis=1) / 2.0
    mel = jnp.mean(jnp.maximum(MARGIN - entropy, 0.0))
    lane = jax.lax.broadcasted_iota(jnp.int32, (1, 1, 1, 128), 3)
    stats_ref[...] = jnp.where(lane == 0, max_mu,
                               jnp.where(lane == 1, max_ls, mel))


def _encode(x_all, weights, noise):
    grid = (2, B)
    wspec = pl.BlockSpec((1, D, D), lambda m, b: (m, 0, 0))
    out = pl.pallas_call(
        _enc_kernel,
        grid=grid,
        in_specs=[
            pl.BlockSpec((1, 1, S, D), lambda m, b: (m, b, 0, 0)),
            wspec, wspec, wspec, wspec, wspec, wspec,
            pl.BlockSpec((1, 1, SAMPLE_NUM, S, D), lambda m, b: (m, b, 0, 0, 0)),
        ],
        out_specs=[
            pl.BlockSpec((1, 1, N1, D), lambda m, b: (m, b, 0, 0)),
            pl.BlockSpec((1, 1, 1, 128), lambda m, b: (m, b, 0, 0)),
        ],
        out_shape=[
            jax.ShapeDtypeStruct((2, B, N1, D), jnp.float32),
            jax.ShapeDtypeStruct((2, B, 1, 128), jnp.float32),
        ],
    )(x_all, *weights, noise)
    return out


def _top16_thr(sim):
    thr = jnp.max(sim, axis=1, keepdims=True)
    for _ in range(K_NN - 1):
        thr = jnp.max(jnp.where(sim < thr, sim, -jnp.inf), axis=1, keepdims=True)
    return thr


def _gcn1_kernel(nodes_ref, w1m_ref, w1te_ref,
                 h1m_ref, maskm_ref, h1te_ref, maskte_ref):
    nb = pl.program_id(1)
    nodes = nodes_ref[0]
    rows = nodes_ref[0, pl.ds(nb * BR, BR), :]
    sim = jax.lax.dot_general(rows, nodes, (((1,), (1,)), ((), ())),
                              preferred_element_type=jnp.float32)
    nodes_bf = nodes.astype(jnp.bfloat16)

    mask_m = (sim >= _top16_thr(sim)).astype(jnp.bfloat16)
    maskm_ref[0] = mask_m
    agg_m = jax.lax.dot_general(mask_m, nodes_bf, (((1,), (0,)), ((), ())),
                                preferred_element_type=jnp.float32) * (1.0 / K_NN)
    h1m_ref[0] = jnp.maximum(agg_m @ w1m_ref[0], 0.0).astype(jnp.bfloat16)

    base = jnp.where(nb < NB // 2, 0, N1)
    sim_sub = jnp.where(nb < NB // 2, sim[:, :N1], sim[:, N1:])
    half_nodes = nodes_ref[0, pl.ds(base, N1), :].astype(jnp.bfloat16)
    mask_te = (sim_sub >= _top16_thr(sim_sub)).astype(jnp.bfloat16)
    maskte_ref[0] = mask_te
    agg_te = jax.lax.dot_general(mask_te, half_nodes, (((1,), (0,)), ((), ())),
                                 preferred_element_type=jnp.float32) * (1.0 / K_NN)
    h1te_ref[0] = jnp.maximum(agg_te @ w1te_ref[0], 0.0).astype(jnp.bfloat16)


def _gcn2_kernel(maskm_ref, h1m_ref, maskte_ref, h1te_ref,
                 w2m_ref, wcm_ref, w2te_ref, wcte_ref,
                 outm_ref, outte_ref, accm_ref, accte_ref):
    nb = pl.program_id(1)

    @pl.when(nb == 0)
    def _():
        accm_ref[...] = jnp.zeros_like(accm_ref)

    @pl.when((nb == 0) | (nb == NB // 2))
    def _():
        accte_ref[...] = jnp.zeros_like(accte_ref)

    agg2_m = jax.lax.dot_general(maskm_ref[0], h1m_ref[0], (((1,), (0,)), ((), ())),
                                 preferred_element_type=jnp.float32) * (1.0 / K_NN)
    h2_m = jnp.maximum(agg2_m @ w2m_ref[0], 0.0)
    accm_ref[...] += jnp.sum(h2_m, axis=0, keepdims=True)

    agg2_te = jax.lax.dot_general(maskte_ref[0], h1te_ref[0], (((1,), (0,)), ((), ())),
                                  preferred_element_type=jnp.float32) * (1.0 / K_NN)
    h2_te = jnp.maximum(agg2_te @ w2te_ref[0], 0.0)
    accte_ref[...] += jnp.sum(h2_te, axis=0, keepdims=True)

    @pl.when(nb == NB // 2 - 1)
    def _():
        outte_ref[0, 0:1, :] = (accte_ref[...] * (1.0 / N1)) @ wcte_ref[0]

    @pl.when(nb == NB - 1)
    def _():
        outte_ref[0, 1:2, :] = (accte_ref[...] * (1.0 / N1)) @ wcte_ref[0]
        outm_ref[0] = (accm_ref[...] * (1.0 / N2)) @ wcm_ref[0]


def _gcn_all(m_nodes, w1m, w2m, wcm, w1te, w2te, wcte):
    half = lambda b, nb: (nb // (NB // 2), 0, 0)
    h1m, maskm, h1te, maskte = pl.pallas_call(
        _gcn1_kernel,
        grid=(B, NB),
        in_specs=[
            pl.BlockSpec((1, N2, D), lambda b, nb: (b, 0, 0)),
            pl.BlockSpec((1, D, 512), lambda b, nb: (0, 0, 0)),
            pl.BlockSpec((1, D, 512), half),
        ],
        out_specs=[
            pl.BlockSpec((1, BR, 512), lambda b, nb: (b, nb, 0)),
            pl.BlockSpec((1, BR, N2), lambda b, nb: (b, nb, 0)),
            pl.BlockSpec((1, BR, 512), lambda b, nb: (b, nb, 0)),
            pl.BlockSpec((1, BR, N1), lambda b, nb: (b, nb, 0)),
        ],
        out_shape=[
            jax.ShapeDtypeStruct((B, N2, 512), jnp.bfloat16),
            jax.ShapeDtypeStruct((B, N2, N2), jnp.bfloat16),
            jax.ShapeDtypeStruct((B, N2, 512), jnp.bfloat16),
            jax.ShapeDtypeStruct((B, N2, N1), jnp.bfloat16),
        ],
    )(m_nodes, w1m[None], jnp.stack([w1te[0], w1te[1]]))
    outm, outte = pl.pallas_call(
        _gcn2_kernel,
        grid=(B, NB),
        in_specs=[
            pl.BlockSpec((1, BR, N2), lambda b, nb: (b, nb, 0)),
            pl.BlockSpec((1, N2, 512), lambda b, nb: (b, 0, 0)),
            pl.BlockSpec((1, BR, N1), lambda b, nb: (b, nb, 0)),
            pl.BlockSpec((1, N1, 512), lambda b, nb: (b, nb // (NB // 2), 0)),
            pl.BlockSpec((1, 512, 256), lambda b, nb: (0, 0, 0)),
            pl.BlockSpec((1, 256, 2), lambda b, nb: (0, 0, 0)),
            pl.BlockSpec((1, 512, 256), half),
            pl.BlockSpec((1, 256, 2), half),
        ],
        out_specs=[
            pl.BlockSpec((1, 1, 2), lambda b, nb: (b, 0, 0)),
            pl.BlockSpec((1, 2, 2), lambda b, nb: (b, 0, 0)),
        ],
        out_shape=[
            jax.ShapeDtypeStruct((B, 1, 2), jnp.float32),
            jax.ShapeDtypeStruct((B, 2, 2), jnp.float32),
        ],
        scratch_shapes=[pltpu.VMEM((1, 256), jnp.float32),
                        pltpu.VMEM((1, 256), jnp.float32)],
    )(maskm, h1m, maskte, h1te, w2m[None], wcm[None],
      jnp.stack([w2te[0], w2te[1]]), jnp.stack([wcte[0], wcte[1]]))
    return outm[:, 0, :], outte


def kernel(img_embeds, text_embeds,
           img_Wq, img_Wk, img_Wv, img_Wo, img_Wmu, img_Wls,
           txt_Wq, txt_Wk, txt_Wv, txt_Wo, txt_Wmu, txt_Wls,
           tcls_W1, tcls_W2, tcls_Wc,
           icls_W1, icls_W2, icls_Wc,
           hcls_W1, hcls_W2, hcls_Wc):
    x_all = jnp.stack([img_embeds, text_embeds])
    weights = [jnp.stack([i, t]) for i, t in [
        (img_Wq, txt_Wq), (img_Wk, txt_Wk), (img_Wv, txt_Wv),
        (img_Wo, txt_Wo), (img_Wmu, txt_Wmu), (img_Wls, txt_Wls)]]
    noise = jnp.asarray(_NOISE)
    nodes, stats = _encode(x_all, weights, noise)
    m_nodes = jnp.concatenate([nodes[1], nodes[0]], axis=1)

    deci_m, deci_te = _gcn_all(
        m_nodes, hcls_W1, hcls_W2, hcls_Wc,
        (tcls_W1, icls_W1), (tcls_W2, icls_W2), (tcls_Wc, icls_Wc))
    deci_t, deci_e = deci_te[:, 0, :], deci_te[:, 1, :]

    max_mu = stats[..., 0, 0]
    max_ls = stats[..., 0, 1]
    mel = stats[..., 0, 2]
    std = jnp.sqrt(jnp.maximum(max_ls, 0.0) + 1e-06)
    score = jax.nn.sigmoid(std / max_mu)
    img_score, txt_score = score[0], score[1]
    m_ab = txt_score * img_score
    margin_loss = (mel[0, B - 1] + mel[1, B - 1]) / 2.0
    preds = jnp.where((m_ab > GAMMA)[:, None], deci_m,
                      jnp.where((txt_score > img_score)[:, None], deci_t, deci_e))
    return preds, margin_loss


# speedup vs baseline: 1.2231x; 1.0093x over previous
import functools
import math

import jax
import jax.numpy as jnp
import numpy as np
from jax.experimental import pallas as pl
from jax.experimental.pallas import tpu as pltpu

B, S, D, H = 2, 512, 768, 12
SAMPLE_NUM, MU_NUM, K_NN = 2, 1, 16
MARGIN, GAMMA = 10.0, 0.5
DH = D // H
N1 = (MU_NUM + SAMPLE_NUM) * S
N2 = 2 * N1
BR = 256
NB = N2 // BR
_ENT_CONST = D / 2.0 * (math.log(2.0 * math.pi) + 1.0)


def _make_noise() -> np.ndarray:
    key = jax.random.key(42)
    out = np.zeros((2, B, SAMPLE_NUM, S, D), np.float32)
    for b in range(B):
        for s in range(SAMPLE_NUM):
            for m in range(2):
                k = jax.random.fold_in(key, b * 100 + s + 50 * m)
                out[m, b, s] = np.asarray(jax.random.normal(k, (S, D), jnp.float32))
    return out


_NOISE = _make_noise()


def _enc_kernel(x_ref, wq_ref, wk_ref, wv_ref, wo_ref, wmu_ref, wls_ref,
                n_ref, nodes_ref, stats_ref):
    x = x_ref[0, 0]
    q = x @ wq_ref[0]
    k = x @ wk_ref[0]
    v = x @ wv_ref[0]
    scale = 1.0 / math.sqrt(DH)
    heads = []
    for h in range(H):
        qh = q[:, h * DH:(h + 1) * DH]
        kh = k[:, h * DH:(h + 1) * DH]
        vh = v[:, h * DH:(h + 1) * DH]
        logits = jax.lax.dot_general(qh, kh, (((1,), (1,)), ((), ())),
                                     preferred_element_type=jnp.float32) * scale
        m = jnp.max(logits, axis=-1, keepdims=True)
        e = jnp.exp(logits - m)
        att = e / jnp.sum(e, axis=-1, keepdims=True)
        heads.append(att @ vh)
    hid = jnp.concatenate(heads, axis=1) @ wo_ref[0] + x
    mu = hid @ wmu_ref[0]
    ls = hid @ wls_ref[0]
    e_ls = jnp.exp(ls)
    nodes_ref[0, 0, 0:S, :] = mu
    nodes_ref[0, 0, S:2 * S, :] = mu + e_ls * n_ref[0, 0, 0]
    nodes_ref[0, 0, 2 * S:3 * S, :] = mu + e_ls * n_ref[0, 0, 1]
    max_mu = jnp.max(mu)
    max_ls = jnp.max(ls)
    entropy = _ENT_CONST + jnp.sum(ls, axis=1) / 2.0
    mel = jnp.mean(jnp.maximum(MARGIN - entropy, 0.0))
    lane = jax.lax.broadcasted_iota(jnp.int32, (1, 1, 1, 128), 3)
    stats_ref[...] = jnp.where(lane == 0, max_mu,
                               jnp.where(lane == 1, max_ls, mel))


def _encode(x_all, weights, noise):
    grid = (2, B)
    wspec = pl.BlockSpec((1, D, D), lambda m, b: (m, 0, 0))
    out = pl.pallas_call(
        _enc_kernel,
        grid=grid,
        in_specs=[
            pl.BlockSpec((1, 1, S, D), lambda m, b: (m, b, 0, 0)),
            wspec, wspec, wspec, wspec, wspec, wspec,
            pl.BlockSpec((1, 1, SAMPLE_NUM, S, D), lambda m, b: (m, b, 0, 0, 0)),
        ],
        out_specs=[
            pl.BlockSpec((1, 1, N1, D), lambda m, b: (m, b, 0, 0)),
            pl.BlockSpec((1, 1, 1, 128), lambda m, b: (m, b, 0, 0)),
        ],
        out_shape=[
            jax.ShapeDtypeStruct((2, B, N1, D), jnp.float32),
            jax.ShapeDtypeStruct((2, B, 1, 128), jnp.float32),
        ],
    )(x_all, *weights, noise)
    return out


def _top16_thr(sim):
    thr = jnp.max(sim, axis=1, keepdims=True)
    for _ in range(K_NN - 1):
        thr = jnp.max(jnp.where(sim < thr, sim, -jnp.inf), axis=1, keepdims=True)
    return thr


def _gcn_kernel(nodes_ref, w1m_ref, w1te_ref, w2m_ref, wcm_ref,
                w2te_ref, wcte_ref, outm_ref, outte_ref,
                maskm_s, maskte_s, h1m_s, h1te_s, accm_ref, accte_ref):
    ph = pl.program_id(1)
    nb = pl.program_id(2)

    @pl.when(ph == 0)
    def _phase1():
        nodes = nodes_ref[0]
        rows = nodes_ref[0, pl.ds(nb * BR, BR), :]
        sim = jax.lax.dot_general(rows, nodes, (((1,), (1,)), ((), ())),
                                  preferred_element_type=jnp.float32)
        nodes_bf = nodes.astype(jnp.bfloat16)

        mask_m = sim >= _top16_thr(sim)
        maskm_s[pl.ds(nb * BR, BR), :] = mask_m.astype(jnp.int8)
        agg_m = jax.lax.dot_general(mask_m.astype(jnp.bfloat16), nodes_bf,
                                    (((1,), (0,)), ((), ())),
                                    preferred_element_type=jnp.float32) * (1.0 / K_NN)
        h1m_s[pl.ds(nb * BR, BR), :] = (
            jnp.maximum(agg_m @ w1m_ref[0], 0.0).astype(jnp.bfloat16))

        base = jnp.where(nb < NB // 2, 0, N1)
        sim_sub = jnp.where(nb < NB // 2, sim[:, :N1], sim[:, N1:])
        half_nodes = nodes_ref[0, pl.ds(base, N1), :].astype(jnp.bfloat16)
        mask_te = sim_sub >= _top16_thr(sim_sub)
        maskte_s[pl.ds(nb * BR, BR), :] = mask_te.astype(jnp.int8)
        agg_te = jax.lax.dot_general(mask_te.astype(jnp.bfloat16), half_nodes,
                                     (((1,), (0,)), ((), ())),
                                     preferred_element_type=jnp.float32) * (1.0 / K_NN)
        h1te_s[pl.ds(nb * BR, BR), :] = (
            jnp.maximum(agg_te @ w1te_ref[0], 0.0).astype(jnp.bfloat16))

    @pl.when(ph == 1)
    def _phase2():
        @pl.when(nb == 0)
        def _():
            accm_ref[...] = jnp.zeros_like(accm_ref)

        @pl.when((nb == 0) | (nb == NB // 2))
        def _():
            accte_ref[...] = jnp.zeros_like(accte_ref)

        mask_m = maskm_s[pl.ds(nb * BR, BR), :].astype(jnp.bfloat16)
        agg2_m = jax.lax.dot_general(mask_m, h1m_s[...], (((1,), (0,)), ((), ())),
                                     preferred_element_type=jnp.float32) * (1.0 / K_NN)
        h2_m = jnp.maximum(agg2_m @ w2m_ref[0], 0.0)
        accm_ref[...] += jnp.sum(h2_m, axis=0, keepdims=True)

        base = jnp.where(nb < NB // 2, 0, N1)
        mask_te = maskte_s[pl.ds(nb * BR, BR), :].astype(jnp.bfloat16)
        h1_half = h1te_s[pl.ds(base, N1), :]
        agg2_te = jax.lax.dot_general(mask_te, h1_half, (((1,), (0,)), ((), ())),
                                      preferred_element_type=jnp.float32) * (1.0 / K_NN)
        h2_te = jnp.maximum(agg2_te @ w2te_ref[0], 0.0)
        accte_ref[...] += jnp.sum(h2_te, axis=0, keepdims=True)

        @pl.when(nb == NB // 2 - 1)
        def _():
            outte_ref[0, 0:1, :] = (accte_ref[...] * (1.0 / N1)) @ wcte_ref[0]

        @pl.when(nb == NB - 1)
        def _():
            outte_ref[0, 1:2, :] = (accte_ref[...] * (1.0 / N1)) @ wcte_ref[0]
            outm_ref[0] = (accm_ref[...] * (1.0 / N2)) @ wcm_ref[0]


def _gcn_all(m_nodes, w1m, w2m, wcm, w1te, w2te, wcte):
    half = lambda b, ph, nb: (nb // (NB // 2), 0, 0)
    const = lambda b, ph, nb: (0, 0, 0)
    outm, outte = pl.pallas_call(
        _gcn_kernel,
        grid=(B, 2, NB),
        in_specs=[
            pl.BlockSpec((1, N2, D), lambda b, ph, nb: (b, 0, 0)),
            pl.BlockSpec((1, D, 512), const),
            pl.BlockSpec((1, D, 512), half),
            pl.BlockSpec((1, 512, 256), const),
            pl.BlockSpec((1, 256, 2), const),
            pl.BlockSpec((1, 512, 256), half),
            pl.BlockSpec((1, 256, 2), half),
        ],
        out_specs=[
            pl.BlockSpec((1, 1, 2), lambda b, ph, nb: (b, 0, 0)),
            pl.BlockSpec((1, 2, 2), lambda b, ph, nb: (b, 0, 0)),
        ],
        out_shape=[
            jax.ShapeDtypeStruct((B, 1, 2), jnp.float32),
            jax.ShapeDtypeStruct((B, 2, 2), jnp.float32),
        ],
        scratch_shapes=[
            pltpu.VMEM((N2, N2), jnp.int8),
            pltpu.VMEM((N2, N1), jnp.int8),
            pltpu.VMEM((N2, 512), jnp.bfloat16),
            pltpu.VMEM((N2, 512), jnp.bfloat16),
            pltpu.VMEM((1, 256), jnp.float32),
            pltpu.VMEM((1, 256), jnp.float32),
        ],
    )(m_nodes, w1m[None], jnp.stack([w1te[0], w1te[1]]),
      w2m[None], wcm[None],
      jnp.stack([w2te[0], w2te[1]]), jnp.stack([wcte[0], wcte[1]]))
    return outm[:, 0, :], outte


def kernel(img_embeds, text_embeds,
           img_Wq, img_Wk, img_Wv, img_Wo, img_Wmu, img_Wls,
           txt_Wq, txt_Wk, txt_Wv, txt_Wo, txt_Wmu, txt_Wls,
           tcls_W1, tcls_W2, tcls_Wc,
           icls_W1, icls_W2, icls_Wc,
           hcls_W1, hcls_W2, hcls_Wc):
    x_all = jnp.stack([img_embeds, text_embeds])
    weights = [jnp.stack([i, t]) for i, t in [
        (img_Wq, txt_Wq), (img_Wk, txt_Wk), (img_Wv, txt_Wv),
        (img_Wo, txt_Wo), (img_Wmu, txt_Wmu), (img_Wls, txt_Wls)]]
    noise = jnp.asarray(_NOISE)
    nodes, stats = _encode(x_all, weights, noise)
    m_nodes = jnp.concatenate([nodes[1], nodes[0]], axis=1)

    deci_m, deci_te = _gcn_all(
        m_nodes, hcls_W1, hcls_W2, hcls_Wc,
        (tcls_W1, icls_W1), (tcls_W2, icls_W2), (tcls_Wc, icls_Wc))
    deci_t, deci_e = deci_te[:, 0, :], deci_te[:, 1, :]

    max_mu = stats[..., 0, 0]
    max_ls = stats[..., 0, 1]
    mel = stats[..., 0, 2]
    std = jnp.sqrt(jnp.maximum(max_ls, 0.0) + 1e-06)
    score = jax.nn.sigmoid(std / max_mu)
    img_score, txt_score = score[0], score[1]
    m_ab = txt_score * img_score
    margin_loss = (mel[0, B - 1] + mel[1, B - 1]) / 2.0
    preds = jnp.where((m_ab > GAMMA)[:, None], deci_m,
                      jnp.where((txt_score > img_score)[:, None], deci_t, deci_e))
    return preds, margin_loss


# speedup vs baseline: 1.2233x; 1.0001x over previous
import functools
import math

import jax
import jax.numpy as jnp
import numpy as np
from jax.experimental import pallas as pl
from jax.experimental.pallas import tpu as pltpu

B, S, D, H = 2, 512, 768, 12
SAMPLE_NUM, MU_NUM, K_NN = 2, 1, 16
MARGIN, GAMMA = 10.0, 0.5
DH = D // H
N1 = (MU_NUM + SAMPLE_NUM) * S
N2 = 2 * N1
BR = 256
NB = N2 // BR
_ENT_CONST = D / 2.0 * (math.log(2.0 * math.pi) + 1.0)


def _make_noise() -> np.ndarray:
    key = jax.random.key(42)
    out = np.zeros((2, B, SAMPLE_NUM, S, D), np.float32)
    for b in range(B):
        for s in range(SAMPLE_NUM):
            for m in range(2):
                k = jax.random.fold_in(key, b * 100 + s + 50 * m)
                out[m, b, s] = np.asarray(jax.random.normal(k, (S, D), jnp.float32))
    return out


_NOISE = _make_noise()


def _enc_kernel(x_ref, wq_ref, wk_ref, wv_ref, wo_ref, wmu_ref, wls_ref,
                n_ref, nodes_ref, stats_ref):
    x = x_ref[0, 0]
    q = x @ wq_ref[0]
    k = x @ wk_ref[0]
    v = x @ wv_ref[0]
    scale = 1.0 / math.sqrt(DH)
    heads = []
    for h in range(H):
        qh = q[:, h * DH:(h + 1) * DH]
        kh = k[:, h * DH:(h + 1) * DH]
        vh = v[:, h * DH:(h + 1) * DH]
        logits = jax.lax.dot_general(qh, kh, (((1,), (1,)), ((), ())),
                                     preferred_element_type=jnp.float32) * scale
        m = jnp.max(logits, axis=-1, keepdims=True)
        e = jnp.exp(logits - m)
        att = e / jnp.sum(e, axis=-1, keepdims=True)
        heads.append(att @ vh)
    hid = jnp.concatenate(heads, axis=1) @ wo_ref[0] + x
    mu = hid @ wmu_ref[0]
    ls = hid @ wls_ref[0]
    e_ls = jnp.exp(ls)
    nodes_ref[0, 0, 0:S, :] = mu
    nodes_ref[0, 0, S:2 * S, :] = mu + e_ls * n_ref[0, 0, 0]
    nodes_ref[0, 0, 2 * S:3 * S, :] = mu + e_ls * n_ref[0, 0, 1]
    max_mu = jnp.max(mu)
    max_ls = jnp.max(ls)
    entropy = _ENT_CONST + jnp.sum(ls, axis=1) / 2.0
    mel = jnp.mean(jnp.maximum(MARGIN - entropy, 0.0))
    lane = jax.lax.broadcasted_iota(jnp.int32, (1, 1, 1, 128), 3)
    stats_ref[...] = jnp.where(lane == 0, max_mu,
                               jnp.where(lane == 1, max_ls, mel))


def _encode(x_all, weights, noise):
    grid = (2, B)
    wspec = pl.BlockSpec((1, D, D), lambda m, b: (m, 0, 0))
    out = pl.pallas_call(
        _enc_kernel,
        grid=grid,
        in_specs=[
            pl.BlockSpec((1, 1, S, D), lambda m, b: (m, b, 0, 0)),
            wspec, wspec, wspec, wspec, wspec, wspec,
            pl.BlockSpec((1, 1, SAMPLE_NUM, S, D), lambda m, b: (m, b, 0, 0, 0)),
        ],
        out_specs=[
            pl.BlockSpec((1, 1, N1, D), lambda m, b: (m, b, 0, 0)),
            pl.BlockSpec((1, 1, 1, 128), lambda m, b: (m, b, 0, 0)),
        ],
        out_shape=[
            jax.ShapeDtypeStruct((2, B, N1, D), jnp.float32),
            jax.ShapeDtypeStruct((2, B, 1, 128), jnp.float32),
        ],
    )(x_all, *weights, noise)
    return out


def _top16_thr(sim):
    thr = jnp.max(sim, axis=1, keepdims=True)
    for _ in range(K_NN - 1):
        thr = jnp.max(jnp.where(sim < thr, sim, -jnp.inf), axis=1, keepdims=True)
    return thr


def _gcn_kernel(nodes_ref, w1m_ref, w1te_ref, w2m_ref, wcm_ref,
                w2te_ref, wcte_ref, outm_ref, outte_ref,
                maskm_s, maskte_s, h1m_s, h1te_s, accm_ref, accte_ref):
    ph = pl.program_id(1)
    nb = pl.program_id(2)

    @pl.when(ph == 0)
    def _phase1():
        nodes = nodes_ref[0]
        rows = nodes_ref[0, pl.ds(nb * BR, BR), :]
        sim = jax.lax.dot_general(rows, nodes, (((1,), (1,)), ((), ())),
                                  preferred_element_type=jnp.float32)
        nodes_bf = nodes.astype(jnp.bfloat16)

        mask_m = sim >= _top16_thr(sim)
        maskm_s[pl.ds(nb * BR, BR), :] = mask_m.astype(jnp.int8)
        agg_m = jax.lax.dot_general(mask_m.astype(jnp.bfloat16), nodes_bf,
                                    (((1,), (0,)), ((), ())),
                                    preferred_element_type=jnp.float32) * (1.0 / K_NN)
        h1m_s[pl.ds(nb * BR, BR), :] = jnp.maximum(jax.lax.dot_general(
            agg_m.astype(jnp.bfloat16), w1m_ref[0].astype(jnp.bfloat16),
            (((1,), (0,)), ((), ())), preferred_element_type=jnp.float32),
            0.0).astype(jnp.bfloat16)

        base = jnp.where(nb < NB // 2, 0, N1)
        sim_sub = jnp.where(nb < NB // 2, sim[:, :N1], sim[:, N1:])
        half_nodes = nodes_ref[0, pl.ds(base, N1), :].astype(jnp.bfloat16)
        mask_te = sim_sub >= _top16_thr(sim_sub)
        maskte_s[pl.ds(nb * BR, BR), :] = mask_te.astype(jnp.int8)
        agg_te = jax.lax.dot_general(mask_te.astype(jnp.bfloat16), half_nodes,
                                     (((1,), (0,)), ((), ())),
                                     preferred_element_type=jnp.float32) * (1.0 / K_NN)
        h1te_s[pl.ds(nb * BR, BR), :] = jnp.maximum(jax.lax.dot_general(
            agg_te.astype(jnp.bfloat16), w1te_ref[0].astype(jnp.bfloat16),
            (((1,), (0,)), ((), ())), preferred_element_type=jnp.float32),
            0.0).astype(jnp.bfloat16)

    @pl.when(ph == 1)
    def _phase2():
        @pl.when(nb == 0)
        def _():
            accm_ref[...] = jnp.zeros_like(accm_ref)

        @pl.when((nb == 0) | (nb == NB // 2))
        def _():
            accte_ref[...] = jnp.zeros_like(accte_ref)

        mask_m = maskm_s[pl.ds(nb * BR, BR), :].astype(jnp.bfloat16)
        agg2_m = jax.lax.dot_general(mask_m, h1m_s[...], (((1,), (0,)), ((), ())),
                                     preferred_element_type=jnp.float32) * (1.0 / K_NN)
        h2_m = jnp.maximum(jax.lax.dot_general(
            agg2_m.astype(jnp.bfloat16), w2m_ref[0].astype(jnp.bfloat16),
            (((1,), (0,)), ((), ())), preferred_element_type=jnp.float32), 0.0)
        accm_ref[...] += jnp.sum(h2_m, axis=0, keepdims=True)

        base = jnp.where(nb < NB // 2, 0, N1)
        mask_te = maskte_s[pl.ds(nb * BR, BR), :].astype(jnp.bfloat16)
        h1_half = h1te_s[pl.ds(base, N1), :]
        agg2_te = jax.lax.dot_general(mask_te, h1_half, (((1,), (0,)), ((), ())),
                                      preferred_element_type=jnp.float32) * (1.0 / K_NN)
        h2_te = jnp.maximum(jax.lax.dot_general(
            agg2_te.astype(jnp.bfloat16), w2te_ref[0].astype(jnp.bfloat16),
            (((1,), (0,)), ((), ())), preferred_element_type=jnp.float32), 0.0)
        accte_ref[...] += jnp.sum(h2_te, axis=0, keepdims=True)

        @pl.when(nb == NB // 2 - 1)
        def _():
            outte_ref[0, 0:1, :] = (accte_ref[...] * (1.0 / N1)) @ wcte_ref[0]

        @pl.when(nb == NB - 1)
        def _():
            outte_ref[0, 1:2, :] = (accte_ref[...] * (1.0 / N1)) @ wcte_ref[0]
            outm_ref[0] = (accm_ref[...] * (1.0 / N2)) @ wcm_ref[0]


def _gcn_all(m_nodes, w1m, w2m, wcm, w1te, w2te, wcte):
    half = lambda b, ph, nb: (nb // (NB // 2), 0, 0)
    const = lambda b, ph, nb: (0, 0, 0)
    outm, outte = pl.pallas_call(
        _gcn_kernel,
        grid=(B, 2, NB),
        in_specs=[
            pl.BlockSpec((1, N2, D), lambda b, ph, nb: (b, 0, 0)),
            pl.BlockSpec((1, D, 512), const),
            pl.BlockSpec((1, D, 512), half),
            pl.BlockSpec((1, 512, 256), const),
            pl.BlockSpec((1, 256, 2), const),
            pl.BlockSpec((1, 512, 256), half),
            pl.BlockSpec((1, 256, 2), half),
        ],
        out_specs=[
            pl.BlockSpec((1, 1, 2), lambda b, ph, nb: (b, 0, 0)),
            pl.BlockSpec((1, 2, 2), lambda b, ph, nb: (b, 0, 0)),
        ],
        out_shape=[
            jax.ShapeDtypeStruct((B, 1, 2), jnp.float32),
            jax.ShapeDtypeStruct((B, 2, 2), jnp.float32),
        ],
        scratch_shapes=[
            pltpu.VMEM((N2, N2), jnp.int8),
            pltpu.VMEM((N2, N1), jnp.int8),
            pltpu.VMEM((N2, 512), jnp.bfloat16),
            pltpu.VMEM((N2, 512), jnp.bfloat16),
            pltpu.VMEM((1, 256), jnp.float32),
            pltpu.VMEM((1, 256), jnp.float32),
        ],
    )(m_nodes, w1m[None], jnp.stack([w1te[0], w1te[1]]),
      w2m[None], wcm[None],
      jnp.stack([w2te[0], w2te[1]]), jnp.stack([wcte[0], wcte[1]]))
    return outm[:, 0, :], outte


def kernel(img_embeds, text_embeds,
           img_Wq, img_Wk, img_Wv, img_Wo, img_Wmu, img_Wls,
           txt_Wq, txt_Wk, txt_Wv, txt_Wo, txt_Wmu, txt_Wls,
           tcls_W1, tcls_W2, tcls_Wc,
           icls_W1, icls_W2, icls_Wc,
           hcls_W1, hcls_W2, hcls_Wc):
    x_all = jnp.stack([img_embeds, text_embeds])
    weights = [jnp.stack([i, t]) for i, t in [
        (img_Wq, txt_Wq), (img_Wk, txt_Wk), (img_Wv, txt_Wv),
        (img_Wo, txt_Wo), (img_Wmu, txt_Wmu), (img_Wls, txt_Wls)]]
    noise = jnp.asarray(_NOISE)
    nodes, stats = _encode(x_all, weights, noise)
    m_nodes = jnp.concatenate([nodes[1], nodes[0]], axis=1)

    deci_m, deci_te = _gcn_all(
        m_nodes, hcls_W1, hcls_W2, hcls_Wc,
        (tcls_W1, icls_W1), (tcls_W2, icls_W2), (tcls_Wc, icls_Wc))
    deci_t, deci_e = deci_te[:, 0, :], deci_te[:, 1, :]

    max_mu = stats[..., 0, 0]
    max_ls = stats[..., 0, 1]
    mel = stats[..., 0, 2]
    std = jnp.sqrt(jnp.maximum(max_ls, 0.0) + 1e-06)
    score = jax.nn.sigmoid(std / max_mu)
    img_score, txt_score = score[0], score[1]
    m_ab = txt_score * img_score
    margin_loss = (mel[0, B - 1] + mel[1, B - 1]) / 2.0
    preds = jnp.where((m_ab > GAMMA)[:, None], deci_m,
                      jnp.where((txt_score > img_score)[:, None], deci_t, deci_e))
    return preds, margin_loss
